# baseline (speedup 1.0000x reference)
import jax
import jax.numpy as jnp
from jax.experimental import pallas as pl


def _mlp(x, Ws, bs):
    for i in range(len(Ws)):
        x = x @ Ws[i] + bs[i]
        if i < len(Ws) - 1:
            x = jax.nn.relu(x)
    return x


def _seg_max0(vals, idx, n):
    out = jax.ops.segment_max(vals, idx, num_segments=n)
    return jnp.where(jnp.isfinite(out), out, 0.0)


def _pointnet(x, pos, ei, Ws, bs):
    src, dst = ei[0], ei[1]
    m = jnp.concatenate([x[src], pos[src] - pos[dst]], axis=-1)
    m = _mlp(m, Ws, bs)
    return jax.nn.relu(_seg_max0(m, dst, x.shape[0]))


def _sage(xin, ei, Wl, bl, Wr):
    src, dst = ei[0], ei[1]
    n = xin.shape[0]
    s = jax.ops.segment_sum(xin[src], dst, num_segments=n)
    c = jax.ops.segment_sum(jnp.ones((ei.shape[1],), xin.dtype), dst, num_segments=n)
    mean = s / jnp.maximum(c, 1.0)[:, None]
    return mean @ Wl + bl + xin @ Wr


def _final_body(pooled_ref, w_ref, b_ref, out_ref):
    out = pooled_ref[...] @ w_ref[...] + b_ref[...]
    out_ref[...] = jax.nn.log_softmax(out, axis=-1)


def kernel(x_locs, pos_locs, x_clusters, edge_index_ll, edge_index_lc, edge_index_cc, cluster_batch, mlp0_W0, mlp0_b0, mlp0_W1, mlp0_b1, mlp0_W2, mlp0_b2, mlp1_W0, mlp1_b0, mlp1_W1, mlp1_b1, mlp1_W2, mlp1_b2, mlp2_W0, mlp2_b0, mlp2_W1, mlp2_b1, mlp2_W2, mlp2_b2, sage0_Wl, sage0_bl, sage0_Wr, sage1_Wl, sage1_bl, sage1_Wr, sage2_Wl, sage2_bl, sage2_Wr, lin_W, lin_b):
    x = _pointnet(x_locs, pos_locs, edge_index_ll, [mlp0_W0, mlp0_W1, mlp0_W2], [mlp0_b0, mlp0_b1, mlp0_b2])
    x = _pointnet(x, pos_locs, edge_index_ll, [mlp1_W0, mlp1_W1, mlp1_W2], [mlp1_b0, mlp1_b1, mlp1_b2])
    x = _pointnet(x, pos_locs, edge_index_ll, [mlp2_W0, mlp2_W1, mlp2_W2], [mlp2_b0, mlp2_b1, mlp2_b2])
    agg = _seg_max0(x[edge_index_lc[0]], edge_index_lc[1], x_clusters.shape[0])
    xc = jnp.concatenate([x_clusters, agg], axis=-1)
    xc = _sage(xc, edge_index_cc, sage0_Wl, sage0_bl, sage0_Wr)
    xc = _sage(xc, edge_index_cc, sage1_Wl, sage1_bl, sage1_Wr)
    xc = _sage(xc, edge_index_cc, sage2_Wl, sage2_bl, sage2_Wr)
    pooled = _seg_max0(xc, cluster_batch, 64)
    return pl.pallas_call(
        _final_body,
        out_shape=jax.ShapeDtypeStruct((64, 2), jnp.float32),
    )(pooled, lin_W, lin_b)


# speedup vs baseline: 4.5369x; 4.5369x over previous
import functools

import jax
import jax.numpy as jnp
from jax import lax
from jax.experimental import pallas as pl
from jax.experimental.pallas import tpu as pltpu
from jax.experimental.pallas import tpu_sc as plsc

N_LOCS = 100000
N_CLUSTERS = 10000
E_LL = 3200000
E_LC = 100000
E_CC = 160000
N_GRAPHS = 64

NPAD = 102400
NCPAD = 10240

NCORES = 2
NSUB = 16
NW = NCORES * NSUB

NEG_INF = float("-inf")
F32 = jnp.float32

_SC_PARAMS = pltpu.CompilerParams(use_tc_tiling_on_sc=False,
                                  needs_layout_passes=False)


def _mesh():
    return plsc.VectorSubcoreMesh(
        core_axis_name="c", subcore_axis_name="s",
        num_cores=NCORES, num_subcores=NSUB)


def _wid():
    return lax.axis_index("s") * NCORES + lax.axis_index("c")


def _zero_fill(ref, n, value):
    fill = jnp.full((16,), value, F32)

    def z(i, c):
        ref[pl.ds(i * 16, 16)] = fill
        return c

    lax.fori_loop(0, n // 16, z, 0)


_GDN = lax.GatherDimensionNumbers(
    offset_dims=(), collapsed_slice_dims=(0,), start_index_map=(0,))


def _vperm(x, src):
    return lax.gather(x, src[:, None], _GDN, (1,),
                      mode=lax.GatherScatterMode.PROMISE_IN_BOUNDS)


def _lane():
    return lax.iota(jnp.int32, 16)


def _run_last(sidx):
    lane = _lane()
    nxt = _vperm(sidx, jnp.minimum(lane + 1, 15))
    return (nxt != sidx) | (lane == 15)


def _seg_combine_max(sidx, sval):
    lane = _lane()
    for s in (1, 2, 4, 8):
        prv = jnp.maximum(lane - s, 0)
        same = (_vperm(sidx, prv) == sidx) & (lane >= s)
        sval = jnp.where(same, jnp.maximum(sval, _vperm(sval, prv)), sval)
    return sval


def _seg_combine_sum(sidx, sval):
    lane = _lane()
    for s in (1, 2, 4, 8):
        prv = jnp.maximum(lane - s, 0)
        same = (_vperm(sidx, prv) == sidx) & (lane >= s)
        sval = sval + jnp.where(same, _vperm(sval, prv), 0.0)
    return sval


def _rmw_max(acc, sidx, sval, il):
    cur = plsc.load_gather(acc, [sidx], mask=il)
    plsc.store_scatter(acc, [sidx], jnp.maximum(cur, sval), mask=il)


def _make_sort_pre(E):
    B = 2000
    assert E % B == 0
    NCHT = E // B
    TRIPS = -(-NCHT // NW)
    NV = B // 16

    @functools.partial(
        pl.kernel,
        out_type=(jax.ShapeDtypeStruct((E,), jnp.int32),
                  jax.ShapeDtypeStruct((E,), jnp.int32),
                  jax.ShapeDtypeStruct((E,), jnp.int32)),
        mesh=_mesh(),
        scratch_types=[
            pltpu.VMEM((B,), jnp.int32),
            pltpu.VMEM((B,), jnp.int32),
            pltpu.VMEM((B,), jnp.int32),
            pltpu.VMEM((B,), jnp.int32),
        ],
        compiler_params=_SC_PARAMS,
    )
    def k(dstI, sdst, perm, islast, dst_v, sd_v, pm_v, il_v):
        w = _wid()

        def chunk(t, c):
            ci = t * NW + w

            @pl.when(ci < NCHT)
            def _():
                base = ci * B
                pltpu.sync_copy(dstI.at[pl.ds(base, B)], dst_v)

                def vec(v, c2):
                    idx = dst_v[pl.ds(v * 16, 16)]
                    sidx, pm = plsc.sort_key_val(idx, _lane())
                    il = _run_last(sidx)
                    sd_v[pl.ds(v * 16, 16)] = sidx
                    pm_v[pl.ds(v * 16, 16)] = pm
                    il_v[pl.ds(v * 16, 16)] = jnp.where(il, 1, 0)
                    return c2

                lax.fori_loop(0, NV, vec, 0)
                pltpu.sync_copy(sd_v, sdst.at[pl.ds(base, B)])
                pltpu.sync_copy(pm_v, perm.at[pl.ds(base, B)])
                pltpu.sync_copy(il_v, islast.at[pl.ds(base, B)])

            return c

        lax.fori_loop(0, TRIPS, chunk, 0)

    return k


def _make_edge_gather(E):
    EW = E // NW
    B = 2000
    NCH = EW // B

    @functools.partial(
        pl.kernel,
        out_type=(jax.ShapeDtypeStruct((E, 16), F32),
                  jax.ShapeDtypeStruct((E, 16), F32)),
        mesh=_mesh(),
        scratch_types=[
            pltpu.VMEM((B,), jnp.int32),
            pltpu.VMEM((B,), jnp.int32),
            pltpu.VMEM((B, 16), F32),
            pltpu.VMEM((B, 16), F32),
            pltpu.SemaphoreType.DMA,
            pltpu.SemaphoreType.DMA,
        ],
        compiler_params=_SC_PARAMS,
    )
    def k(srcI, dstI, T, outS, outD, src_v, dst_v, ra, rc, s1, s2):
        w = _wid()

        def chunk(ci, c):
            base = w * EW + ci * B
            pltpu.sync_copy(srcI.at[pl.ds(base, B)], src_v)
            pltpu.sync_copy(dstI.at[pl.ds(base, B)], dst_v)
            cpa = pltpu.async_copy(T.at[src_v], ra, s1)
            cpb = pltpu.async_copy(T.at[dst_v], rc, s2)
            cpa.wait()
            cpb.wait()
            pltpu.sync_copy(ra, outS.at[pl.ds(base, B)])
            pltpu.sync_copy(rc, outD.at[pl.ds(base, B)])
            return c

        lax.fori_loop(0, NCH, chunk, 0)

    return k


def _make_scatter_max_ll(E, d, nsh):
    EW = E // nsh
    B = 2000
    NCH = EW // B
    NV = B // 16

    @functools.partial(
        pl.kernel,
        out_type=jax.ShapeDtypeStruct((nsh, d, NPAD), F32),
        mesh=_mesh(),
        scratch_types=[
            pltpu.VMEM((NPAD,), F32),
            pltpu.VMEM((B,), jnp.int32),
            pltpu.VMEM((B,), jnp.int32),
            pltpu.VMEM((B,), jnp.int32),
            pltpu.VMEM((B,), F32),
        ],
        compiler_params=_SC_PARAMS,
    )
    def k(sdstI, permI, ilI, valT, out, acc, sd_v, pm_v, il_v, val_v):
        w = _wid()

        @pl.when(w < nsh * d)
        def _():
            s = w // d
            j = w % d
            _zero_fill(acc, NPAD, 0.0)

            def chunk(ci, c):
                base = s * EW + ci * B
                pltpu.sync_copy(sdstI.at[pl.ds(base, B)], sd_v)
                pltpu.sync_copy(permI.at[pl.ds(base, B)], pm_v)
                pltpu.sync_copy(ilI.at[pl.ds(base, B)], il_v)
                pltpu.sync_copy(valT.at[j, pl.ds(base, B)], val_v)

                def vec(v, c2):
                    sl = pl.ds(v * 16, 16)
                    sidx = sd_v[sl]
                    il = il_v[sl] != 0
                    sval = _vperm(val_v[sl], pm_v[sl])
                    sval = _seg_combine_max(sidx, sval)
                    _rmw_max(acc, sidx, sval, il)
                    return c2

                lax.fori_loop(0, NV, vec, 0)
                return c

            lax.fori_loop(0, NCH, chunk, 0)
            pltpu.sync_copy(acc, out.at[s, j])

    return k


def _make_scatter_max_lc():
    d = 8
    nsh = 2
    EW = E_LC // nsh
    B = 2000
    NCH = EW // B
    NV = B // 16

    @functools.partial(
        pl.kernel,
        out_type=jax.ShapeDtypeStruct((nsh, d, NCPAD), F32),
        mesh=_mesh(),
        scratch_types=[
            pltpu.VMEM((NCPAD,), F32),
            pltpu.VMEM((NCPAD,), F32),
            pltpu.VMEM((B,), jnp.int32),
            pltpu.VMEM((B,), jnp.int32),
            pltpu.VMEM((B,), jnp.int32),
            pltpu.VMEM((B,), jnp.int32),
        ],
        compiler_params=_SC_PARAMS,
    )
    def k(srcI, sdstI, permI, ilI, xTlc, out, tbl, acc, src_v, sd_v, pm_v,
          il_v):
        w = _wid()

        @pl.when(w < nsh * d)
        def _():
            s = w // d
            j = w % d
            pltpu.sync_copy(xTlc.at[j], tbl)
            _zero_fill(acc, NCPAD, NEG_INF)

            def chunk(ci, c):
                base = s * EW + ci * B
                pltpu.sync_copy(srcI.at[pl.ds(base, B)], src_v)
                pltpu.sync_copy(sdstI.at[pl.ds(base, B)], sd_v)
                pltpu.sync_copy(permI.at[pl.ds(base, B)], pm_v)
                pltpu.sync_copy(ilI.at[pl.ds(base, B)], il_v)

                def vec(v, c2):
                    sl = pl.ds(v * 16, 16)
                    sidx = sd_v[sl]
                    il = il_v[sl] != 0
                    val = plsc.load_gather(tbl, [src_v[sl]])
                    sval = _vperm(val, pm_v[sl])
                    sval = _seg_combine_max(sidx, sval)
                    _rmw_max(acc, sidx, sval, il)
                    return c2

                lax.fori_loop(0, NV, vec, 0)
                return c

            lax.fori_loop(0, NCH, chunk, 0)
            pltpu.sync_copy(acc, out.at[s, j])

    return k


def _make_scatter_sum_cc(din, with_count):
    B = 2000
    NCH = E_CC // B
    NV = B // 16
    outs = [jax.ShapeDtypeStruct((din, NCPAD), F32)]
    if with_count:
        outs.append(jax.ShapeDtypeStruct((NCPAD,), F32))

    @functools.partial(
        pl.kernel,
        out_type=tuple(outs),
        mesh=_mesh(),
        scratch_types=[
            pltpu.VMEM((NCPAD,), F32),
            pltpu.VMEM((NCPAD,), F32),
            pltpu.VMEM((B,), jnp.int32),
            pltpu.VMEM((B,), jnp.int32),
            pltpu.VMEM((B,), jnp.int32),
            pltpu.VMEM((B,), jnp.int32),
        ],
        compiler_params=_SC_PARAMS,
    )
    def k(srcI, sdstI, permI, ilI, xcT, *rest):
        if with_count:
            sT, cnt = rest[0], rest[1]
            scr = rest[2:]
        else:
            sT = rest[0]
            scr = rest[1:]
        tbl, acc, src_v, sd_v, pm_v, il_v = scr
        w = _wid()
        ones = jnp.full((16,), 1.0, F32)

        @pl.when(w < din)
        def _():
            pltpu.sync_copy(xcT.at[w], tbl)
            _zero_fill(acc, NCPAD, 0.0)

            def chunk(ci, c):
                base = ci * B
                pltpu.sync_copy(srcI.at[pl.ds(base, B)], src_v)
                pltpu.sync_copy(sdstI.at[pl.ds(base, B)], sd_v)
                pltpu.sync_copy(permI.at[pl.ds(base, B)], pm_v)
                pltpu.sync_copy(ilI.at[pl.ds(base, B)], il_v)

                def vec(v, c2):
                    sl = pl.ds(v * 16, 16)
                    sidx = sd_v[sl]
                    il = il_v[sl] != 0
                    val = plsc.load_gather(tbl, [src_v[sl]])
                    sval = _vperm(val, pm_v[sl])
                    sval = _seg_combine_sum(sidx, sval)
                    plsc.addupdate_scatter(acc, [sidx], sval, mask=il)
                    return c2

                lax.fori_loop(0, NV, vec, 0)
                return c

            lax.fori_loop(0, NCH, chunk, 0)
            pltpu.sync_copy(acc, sT.at[w])

        if with_count:
            @pl.when(w == din)
            def _():
                _zero_fill(acc, NCPAD, 0.0)

                def chunk(ci, c):
                    base = ci * B
                    pltpu.sync_copy(sdstI.at[pl.ds(base, B)], sd_v)
                    pltpu.sync_copy(ilI.at[pl.ds(base, B)], il_v)

                    def vec(v, c2):
                        sl = pl.ds(v * 16, 16)
                        sidx = sd_v[sl]
                        il = il_v[sl] != 0
                        sval = _seg_combine_sum(sidx, ones)
                        plsc.addupdate_scatter(acc, [sidx], sval, mask=il)
                        return c2

                    lax.fori_loop(0, NV, vec, 0)
                    return c

                lax.fori_loop(0, NCH, chunk, 0)
                pltpu.sync_copy(acc, cnt)

    return k


def _make_pool():
    NV = N_CLUSTERS // 16

    @functools.partial(
        pl.kernel,
        out_type=jax.ShapeDtypeStruct((32, N_GRAPHS), F32),
        mesh=_mesh(),
        scratch_types=[
            pltpu.VMEM((NCPAD,), F32),
            pltpu.VMEM((N_CLUSTERS,), jnp.int32),
            pltpu.VMEM((N_GRAPHS,), F32),
        ],
        compiler_params=_SC_PARAMS,
    )
    def k(batI, xT3, out, tbl, bat_v, acc):
        j = _wid()
        pltpu.sync_copy(xT3.at[j], tbl)
        pltpu.sync_copy(batI, bat_v)
        _zero_fill(acc, N_GRAPHS, NEG_INF)

        def vec(v, c):
            idx = bat_v[pl.ds(v * 16, 16)]
            val = tbl[pl.ds(v * 16, 16)]
            il = _run_last(idx)
            sval = _seg_combine_max(idx, val)
            _rmw_max(acc, idx, sval, il)
            return c

        lax.fori_loop(0, NV, vec, 0)
        pltpu.sync_copy(acc, out.at[j])

    return k


def _pack_table(A, C, d):
    if d == 8:
        return jnp.concatenate([A, C], axis=1)
    z = jnp.zeros_like(A[:, : 8 - d])
    return jnp.concatenate([A, z, C, z], axis=1)


def _proj0(x, pos, W0, b0):
    R = 2000
    d = 4

    def body(x_ref, p_ref, w_ref, b_ref, t_ref):
        W = w_ref[...]
        pb = p_ref[...]
        Cb = pb @ W[2:]
        A = x_ref[...] @ W[:2] + Cb + b_ref[...]
        t_ref[...] = _pack_table(A, Cb, d)

    return pl.pallas_call(
        body,
        grid=(N_LOCS // R,),
        in_specs=[
            pl.BlockSpec((R, 2), lambda i: (i, 0)),
            pl.BlockSpec((R, 2), lambda i: (i, 0)),
            pl.BlockSpec((4, d), lambda i: (0, 0)),
            pl.BlockSpec((1, d), lambda i: (0, 0)),
        ],
        out_specs=pl.BlockSpec((R, 16), lambda i: (i, 0)),
        out_shape=jax.ShapeDtypeStruct((N_LOCS, 16), F32),
    )(x, pos, W0, b0)


def _edge_mlp(gS, gD, W1, b1, W2, b2, d):
    R = 3200

    def body(a_ref, c_ref, w1_ref, b1_ref, w2_ref, b2_ref, o_ref):
        h = jnp.maximum(a_ref[:, :d] - c_ref[:, 8:8 + d], 0.0)
        h = jnp.maximum(h @ w1_ref[...] + b1_ref[...], 0.0)
        o = h @ w2_ref[...] + b2_ref[...]
        o_ref[...] = o.T

    return pl.pallas_call(
        body,
        grid=(E_LL // R,),
        in_specs=[
            pl.BlockSpec((R, 16), lambda i: (i, 0)),
            pl.BlockSpec((R, 16), lambda i: (i, 0)),
            pl.BlockSpec((d, d), lambda i: (0, 0)),
            pl.BlockSpec((1, d), lambda i: (0, 0)),
            pl.BlockSpec((d, d), lambda i: (0, 0)),
            pl.BlockSpec((1, d), lambda i: (0, 0)),
        ],
        out_specs=pl.BlockSpec((d, R), lambda i: (0, i)),
        out_shape=jax.ShapeDtypeStruct((d, E_LL), F32),
    )(gS, gD, W1, b1, W2, b2)


def _reduce_proj(partials, pos_pad, W0n, b0n, nsh, d, dn):
    R = 2048

    def body(pp_ref, p_ref, w_ref, b_ref, t_ref):
        x = jnp.max(pp_ref[...], axis=0).T
        W = w_ref[...]
        pb = p_ref[...]
        Cb = pb @ W[d:]
        A = x @ W[:d] + Cb + b_ref[...]
        t_ref[...] = _pack_table(A, Cb, dn)

    return pl.pallas_call(
        body,
        grid=(NPAD // R,),
        in_specs=[
            pl.BlockSpec((nsh, d, R), lambda i: (0, 0, i)),
            pl.BlockSpec((R, 2), lambda i: (i, 0)),
            pl.BlockSpec((d + 2, dn), lambda i: (0, 0)),
            pl.BlockSpec((1, dn), lambda i: (0, 0)),
        ],
        out_specs=pl.BlockSpec((R, 16), lambda i: (i, 0)),
        out_shape=jax.ShapeDtypeStruct((NPAD, 16), F32),
    )(partials, pos_pad, W0n, b0n)


def _reduce_last(partials, nsh, d):
    R = 2048

    def body(pp_ref, o_ref):
        o_ref[...] = jnp.max(pp_ref[...], axis=0)

    return pl.pallas_call(
        body,
        grid=(NPAD // R,),
        in_specs=[pl.BlockSpec((nsh, d, R), lambda i: (0, 0, i))],
        out_specs=pl.BlockSpec((d, R), lambda i: (0, i)),
        out_shape=jax.ShapeDtypeStruct((d, NPAD), F32),
    )(partials)


def _concat_clusters(partials_lc, xc_pad):
    R = 2048

    def body(pp_ref, xc_ref, o_ref):
        agg = jnp.max(pp_ref[...], axis=0)
        agg = jnp.where(agg == NEG_INF, 0.0, agg)
        o_ref[...] = jnp.concatenate([xc_ref[...].T, agg], axis=0)

    return pl.pallas_call(
        body,
        grid=(NCPAD // R,),
        in_specs=[
            pl.BlockSpec((2, 8, R), lambda i: (0, 0, i)),
            pl.BlockSpec((R, 9), lambda i: (i, 0)),
        ],
        out_specs=pl.BlockSpec((17, R), lambda i: (0, i)),
        out_shape=jax.ShapeDtypeStruct((17, NCPAD), F32),
    )(partials_lc, xc_pad)


def _sage_dense(sT, cnt, xcT, Wl, bl, Wr, din, dout):
    R = 2048
    dn = (((0,), (0,)), ((), ()))

    def body(s_ref, c_ref, x_ref, wl_ref, bl_ref, wr_ref, o_ref):
        c = jnp.maximum(c_ref[...], 1.0)
        meanT = s_ref[...] / c
        o_ref[...] = (lax.dot_general(wl_ref[...], meanT, dn) + bl_ref[...]
                      + lax.dot_general(wr_ref[...], x_ref[...], dn))

    return pl.pallas_call(
        body,
        grid=(NCPAD // R,),
        in_specs=[
            pl.BlockSpec((din, R), lambda i: (0, i)),
            pl.BlockSpec((1, R), lambda i: (0, i)),
            pl.BlockSpec((din, R), lambda i: (0, i)),
            pl.BlockSpec((din, dout), lambda i: (0, 0)),
            pl.BlockSpec((dout, 1), lambda i: (0, 0)),
            pl.BlockSpec((din, dout), lambda i: (0, 0)),
        ],
        out_specs=pl.BlockSpec((dout, R), lambda i: (0, i)),
        out_shape=jax.ShapeDtypeStruct((dout, NCPAD), F32),
    )(sT, cnt, xcT, Wl, bl, Wr)


def _final(pooledT, lin_W, lin_b):
    def body(p_ref, w_ref, b_ref, o_ref):
        p = p_ref[...]
        p = jnp.where(p == NEG_INF, 0.0, p)
        o = p.T @ w_ref[...] + b_ref[...]
        m = jnp.max(o, axis=-1, keepdims=True)
        lse = m + jnp.log(jnp.sum(jnp.exp(o - m), axis=-1, keepdims=True))
        o_ref[...] = o - lse

    return pl.pallas_call(
        body,
        out_shape=jax.ShapeDtypeStruct((N_GRAPHS, 2), F32),
    )(pooledT, lin_W, lin_b)


_LL_DIMS = (4, 6, 8)
_LL_NSH = (8, 5, 4)


def kernel(x_locs, pos_locs, x_clusters, edge_index_ll, edge_index_lc,
           edge_index_cc, cluster_batch,
           mlp0_W0, mlp0_b0, mlp0_W1, mlp0_b1, mlp0_W2, mlp0_b2,
           mlp1_W0, mlp1_b0, mlp1_W1, mlp1_b1, mlp1_W2, mlp1_b2,
           mlp2_W0, mlp2_b0, mlp2_W1, mlp2_b1, mlp2_W2, mlp2_b2,
           sage0_Wl, sage0_bl, sage0_Wr, sage1_Wl, sage1_bl, sage1_Wr,
           sage2_Wl, sage2_bl, sage2_Wr, lin_W, lin_b):
    srcLL = edge_index_ll[0]
    dstLL = edge_index_ll[1]
    srcLC = edge_index_lc[0]
    dstLC = edge_index_lc[1]
    srcCC = edge_index_cc[0]
    dstCC = edge_index_cc[1]
    pos_pad = jnp.pad(pos_locs, ((0, NPAD - N_LOCS), (0, 0)))
    xc_pad = jnp.pad(x_clusters, ((0, NCPAD - N_CLUSTERS), (0, 0)))

    mlp_W1 = (mlp0_W1, mlp1_W1, mlp2_W1)
    mlp_b1 = (mlp0_b1.reshape(1, -1), mlp1_b1.reshape(1, -1),
              mlp2_b1.reshape(1, -1))
    mlp_W2 = (mlp0_W2, mlp1_W2, mlp2_W2)
    mlp_b2 = (mlp0_b2.reshape(1, -1), mlp1_b2.reshape(1, -1),
              mlp2_b2.reshape(1, -1))
    next_W0 = (mlp1_W0, mlp2_W0)
    next_b0 = (mlp1_b0.reshape(1, -1), mlp2_b0.reshape(1, -1))

    sdLL, pmLL, ilLL = _make_sort_pre(E_LL)(dstLL)
    sdLC, pmLC, ilLC = _make_sort_pre(E_LC)(dstLC)
    sdCC, pmCC, ilCC = _make_sort_pre(E_CC)(dstCC)

    T = _proj0(x_locs, pos_locs, mlp0_W0, mlp0_b0.reshape(1, -1))
    xT = None
    for l in range(3):
        d = _LL_DIMS[l]
        nsh = _LL_NSH[l]
        gS, gD = _make_edge_gather(E_LL)(srcLL, dstLL, T)
        OT = _edge_mlp(gS, gD, mlp_W1[l], mlp_b1[l],
                       mlp_W2[l], mlp_b2[l], d)
        partials = _make_scatter_max_ll(E_LL, d, nsh)(sdLL, pmLL, ilLL, OT)
        if l < 2:
            T = _reduce_proj(partials, pos_pad, next_W0[l], next_b0[l],
                             nsh, d, _LL_DIMS[l + 1])
        else:
            xT = _reduce_last(partials, nsh, d)

    xT_lc = lax.slice(xT, (0, 0), (8, NCPAD))
    partials_lc = _make_scatter_max_lc()(srcLC, sdLC, pmLC, ilLC, xT_lc)
    xcT = _concat_clusters(partials_lc, xc_pad)

    sage_dims = ((17, 18), (18, 24), (24, 32))
    sage_W = ((sage0_Wl, sage0_bl, sage0_Wr),
              (sage1_Wl, sage1_bl, sage1_Wr),
              (sage2_Wl, sage2_bl, sage2_Wr))
    cnt = None
    for l in range(3):
        din, dout = sage_dims[l]
        if l == 0:
            sT, cnt = _make_scatter_sum_cc(din, True)(srcCC, sdCC, pmCC,
                                                       ilCC, xcT)
        else:
            (sT,) = _make_scatter_sum_cc(din, False)(srcCC, sdCC, pmCC,
                                                     ilCC, xcT)
        Wl, bl, Wr = sage_W[l]
        xcT = _sage_dense(sT, cnt.reshape(1, -1), xcT, Wl,
                          bl.reshape(-1, 1), Wr, din, dout)

    pooledT = _make_pool()(cluster_batch, xcT)
    return _final(pooledT, lin_W, lin_b.reshape(1, -1))
